# baseline (speedup 1.0000x reference)
import jax
import jax.numpy as jnp
from jax import lax
from jax.experimental import pallas as pl
from jax.experimental.pallas import tpu as pltpu
from jax.experimental.pallas import tpu_sc as plsc

D = 128
P = 8
A = 4
DEG = 16
B = 256

NC, NS = 2, 16
NW = NC * NS
PAIRS = 2 * B
PPW = PAIRS // NW
SPP = 1 + DEG
SLOTS_PW = PPW * SPP
ROWS = P * (1 + A)
LANES = 16
COLS = D // LANES
NBUF = 2

KBLK = 1000


def _mm_body(imp_ref, wm_ref, out_ref):
    out_ref[...] = lax.dot_general(
        imp_ref[...], wm_ref[0],
        (((1,), (1,)), ((), ())),
        preferred_element_type=jnp.float32)


def _precompute_tables(Impact, WM):
    K = Impact.shape[0]
    nblk = K // KBLK
    return pl.pallas_call(
        _mm_body,
        grid=(2, nblk),
        in_specs=[
            pl.BlockSpec((KBLK, D), lambda i, j: (j, 0)),
            pl.BlockSpec((1, D, D), lambda i, j: (i, 0, 0)),
        ],
        out_specs=pl.BlockSpec((KBLK, D), lambda i, j: (i * nblk + j, 0)),
        out_shape=jax.ShapeDtypeStruct((2 * K, D), jnp.float32),
    )(Impact, WM)


def _idx_body(cat_hbm, ext_hbm, bn_hbm, bn2_hbm, out_hbm,
              selfs_v, selfs2_v, neigh_v, catidx_v, sem_idx):
    c = lax.axis_index("c")
    s = lax.axis_index("s")
    wid = s * NC + c

    pltpu.sync_copy(bn_hbm.at[pl.ds(wid * PPW, PPW)], selfs_v)
    pltpu.sync_copy(bn2_hbm.at[pl.ds(wid * PPW, PPW)], selfs2_v)
    pltpu.async_copy(ext_hbm.at[selfs_v], neigh_v, sem_idx).wait()

    cps = []
    for g in range(PPW):
        cps.append(pltpu.async_copy(
            cat_hbm.at[selfs2_v.at[g]],
            catidx_v.at[pl.ds(g * SPP, 1)], sem_idx))
        cps.append(pltpu.async_copy(
            cat_hbm.at[neigh_v.at[g]],
            catidx_v.at[pl.ds(g * SPP + 1, DEG)], sem_idx))
    for cp in cps:
        cp.wait()
    pltpu.sync_copy(catidx_v, out_hbm.at[pl.ds(wid * SLOTS_PW, SLOTS_PW)])


def _sc_stage_idx(cat_tbl, ext32, bn):
    mesh = plsc.VectorSubcoreMesh(core_axis_name="c", subcore_axis_name="s")
    return pl.kernel(
        _idx_body,
        out_type=jax.ShapeDtypeStruct((PAIRS * SPP, ROWS), jnp.int32),
        mesh=mesh,
        compiler_params=pltpu.CompilerParams(use_tc_tiling_on_sc=False),
        scratch_types=[
            pltpu.VMEM((PPW,), jnp.int32),
            pltpu.VMEM((PPW, 1), jnp.int32),
            pltpu.VMEM((PPW, DEG), jnp.int32),
            pltpu.VMEM((SLOTS_PW, ROWS), jnp.int32),
            pltpu.SemaphoreType.DMA,
        ],
    )(cat_tbl, ext32, bn, bn.reshape(-1, 1))


def _lane_perm(x, idx):
    dnums = lax.GatherDimensionNumbers(
        offset_dims=(), collapsed_slice_dims=(0,), start_index_map=(0,))
    return lax.gather(x, idx[:, None], dnums, slice_sizes=(1,),
                      mode=lax.GatherScatterMode.PROMISE_IN_BOUNDS)


def _main_body(T_hbm, catidx_hbm, out_hbm,
               catidx_v, rows_v, out_v, sem_idx, *sems):
    c = lax.axis_index("c")
    s = lax.axis_index("s")
    wid = s * NC + c

    pltpu.sync_copy(catidx_hbm.at[pl.ds(wid * SLOTS_PW, SLOTS_PW)], catidx_v)

    def issue(j, b):
        pltpu.async_copy(T_hbm.at[catidx_v.at[j]], rows_v.at[b], sems[b])

    def drain(j, b):
        pltpu.make_async_copy(T_hbm.at[catidx_v.at[j]], rows_v.at[b],
                              sems[b]).wait()

    for b in range(NBUF):
        issue(b, b)

    def compute(j, b):
        rcols = []
        for col in range(COLS):
            sl = pl.ds(col * LANES, LANES)
            r16 = None
            for p in range(P):
                acc = rows_v[b, p, sl]
                base = P + A * p
                for a in range(A):
                    acc = acc + rows_v[b, base + a, sl]
                sp = jnp.maximum(acc, 0.0)
                r16 = sp if r16 is None else r16 + sp
            rcols.append(r16)
        lane = lax.iota(jnp.int32, LANES)
        m16 = rcols[0]
        for t in rcols[1:]:
            m16 = jnp.maximum(m16, t)
        for sh in (1, 2, 4, 8):
            m16 = jnp.maximum(m16, _lane_perm(m16, lane ^ sh))
        es = [jnp.exp(t - m16) for t in rcols]
        s16 = es[0]
        for t in es[1:]:
            s16 = s16 + t
        for sh in (1, 2, 4, 8):
            s16 = s16 + _lane_perm(s16, lane ^ sh)
        inv = 1.0 / s16
        for col in range(COLS):
            out_v[j, pl.ds(col * LANES, LANES)] = es[col] * inv

    def body(i, _):
        for b in range(NBUF):
            j = i * NBUF + b
            drain(j, b)
            nxt = j + NBUF

            @pl.when(nxt < SLOTS_PW)
            def _():
                issue(nxt, b)

            compute(j, b)
        return _

    lax.fori_loop(0, SLOTS_PW // NBUF, body, None)
    pltpu.sync_copy(out_v, out_hbm.at[pl.ds(wid * SLOTS_PW, SLOTS_PW)])


def _sc_main(T, catidx_all):
    mesh = plsc.VectorSubcoreMesh(core_axis_name="c", subcore_axis_name="s")
    return pl.kernel(
        _main_body,
        out_type=jax.ShapeDtypeStruct((PAIRS * SPP, D), jnp.float32),
        mesh=mesh,
        compiler_params=pltpu.CompilerParams(use_tc_tiling_on_sc=True),
        scratch_types=[
            pltpu.VMEM((SLOTS_PW, ROWS), jnp.int32),
            pltpu.VMEM((NBUF, ROWS, D), jnp.float32),
            pltpu.VMEM((SLOTS_PW, D), jnp.float32),
            pltpu.SemaphoreType.DMA,
        ] + [pltpu.SemaphoreType.DMA] * NBUF,
    )(T, catidx_all)


def _post_body(enc_ref, U_ref, V_ref, W1_ref, b1_ref, W2_ref, b2_ref,
               out_ref):
    enc = enc_ref[...]
    e0 = enc[:, 0, :]
    nsum = jnp.sum(enc[:, 1:, :], axis=1)
    r = (lax.dot_general(e0, U_ref[...], (((1,), (1,)), ((), ())),
                         preferred_element_type=jnp.float32)
         + lax.dot_general(nsum, V_ref[...], (((1,), (1,)), ((), ())),
                           preferred_element_type=jnp.float32))
    ext = jax.nn.softmax(jax.nn.relu(r), axis=-1)
    e_i = ext[:B]
    e_j = ext[B:]
    third = jnp.concatenate([e_i * e_j, e_i + e_j], axis=-1)
    h = jax.nn.relu(
        lax.dot_general(third, W1_ref[...], (((1,), (1,)), ((), ())),
                        preferred_element_type=jnp.float32) + b1_ref[...])
    logits = lax.dot_general(h, W2_ref[...], (((1,), (1,)), ((), ())),
                             preferred_element_type=jnp.float32) + b2_ref[...]
    out_ref[...] = jax.nn.softmax(logits, axis=-1)


def _post(enc, U, V, W1, b1, W2, b2):
    return pl.pallas_call(
        _post_body,
        out_shape=jax.ShapeDtypeStruct((B, 2), jnp.float32),
    )(enc, U, V, W1, b1, W2, b2)


@jax.jit
def kernel(batch, internal_node_ids, internal_adj, ext_adj,
           Impact, W, M, U, V, W1, b1, W2, b2):
    K = Impact.shape[0]
    n = ext_adj.shape[0]
    batch = batch.astype(jnp.int32)
    ids32 = internal_node_ids.astype(jnp.int32)
    adj32 = internal_adj.reshape(n, P * A).astype(jnp.int32) + K
    cat_tbl = jnp.concatenate([ids32, adj32], axis=1)
    ext32 = ext_adj.astype(jnp.int32)
    bn = jnp.concatenate([batch[:, 0], batch[:, 1]], axis=0)

    catidx_all = _sc_stage_idx(cat_tbl, ext32, bn)
    T = _precompute_tables(Impact, jnp.stack([W, M]))
    enc = _sc_main(T, catidx_all)
    return _post(enc.reshape(PAIRS, SPP, D), U, V, W1,
                 b1.reshape(1, D), W2, b2.reshape(1, 2))


# speedup vs baseline: 1.1603x; 1.1603x over previous
import jax
import jax.numpy as jnp
from jax import lax
from jax.experimental import pallas as pl
from jax.experimental.pallas import tpu as pltpu
from jax.experimental.pallas import tpu_sc as plsc

D = 128
P = 8
A = 4
DEG = 16
B = 256

NC, NS = 2, 16
NW = NC * NS
PAIRS = 2 * B
PPW = PAIRS // NW
SPP = 1 + DEG
SLOTS_PW = PPW * SPP
ROWS = P * (1 + A)
EXT_OFF = ROWS
LANES = 16
COLS = D // LANES
NBUF = 2

KBLK = 2000


def _mm_body(imp_ref, wm_ref, out_ref):
    out_ref[...] = lax.dot_general(
        imp_ref[...], wm_ref[0],
        (((1,), (1,)), ((), ())),
        preferred_element_type=jnp.float32)


def _precompute_tables(Impact, WM):
    K = Impact.shape[0]
    nblk = K // KBLK
    return pl.pallas_call(
        _mm_body,
        grid=(2, nblk),
        in_specs=[
            pl.BlockSpec((KBLK, D), lambda i, j: (j, 0)),
            pl.BlockSpec((1, D, D), lambda i, j: (i, 0, 0)),
        ],
        out_specs=pl.BlockSpec((KBLK, D), lambda i, j: (i * nblk + j, 0)),
        out_shape=jax.ShapeDtypeStruct((2 * K, D), jnp.float32),
    )(Impact, WM)


def _lane_perm(x, idx):
    dnums = lax.GatherDimensionNumbers(
        offset_dims=(), collapsed_slice_dims=(0,), start_index_map=(0,))
    return lax.gather(x, idx[:, None], dnums, slice_sizes=(1,),
                      mode=lax.GatherScatterMode.PROMISE_IN_BOUNDS)


def _sc_body(T_hbm, node_hbm, bn2_hbm, out_hbm,
             selfs2_v, catidx_v, rows_v, out_v,
             sem_idx, *sems):
    c = lax.axis_index("c")
    s = lax.axis_index("s")
    wid = s * NC + c

    pltpu.sync_copy(bn2_hbm.at[pl.ds(wid * PPW, PPW)], selfs2_v)

    cps = [pltpu.async_copy(node_hbm.at[selfs2_v.at[g]],
                            catidx_v.at[pl.ds(g * SPP, 1)], sem_idx)
           for g in range(PPW)]
    for cp in cps:
        cp.wait()
    cps = [pltpu.async_copy(
               node_hbm.at[catidx_v.at[g * SPP, pl.ds(EXT_OFF, DEG)]],
               catidx_v.at[pl.ds(g * SPP + 1, DEG)], sem_idx)
           for g in range(PPW)]
    for cp in cps:
        cp.wait()

    def issue(j, b):
        pltpu.async_copy(T_hbm.at[catidx_v.at[j, pl.ds(0, ROWS)]],
                         rows_v.at[b], sems[b])

    def drain(j, b):
        pltpu.make_async_copy(T_hbm.at[catidx_v.at[j, pl.ds(0, ROWS)]],
                              rows_v.at[b], sems[b]).wait()

    for b in range(NBUF):
        issue(b, b)

    def compute(j, b):
        rcols = []
        for col in range(COLS):
            sl = pl.ds(col * LANES, LANES)
            r16 = None
            for p in range(P):
                acc = rows_v[b, p, sl]
                base = P + A * p
                for a in range(A):
                    acc = acc + rows_v[b, base + a, sl]
                sp = jnp.maximum(acc, 0.0)
                r16 = sp if r16 is None else r16 + sp
            rcols.append(r16)
        lane = lax.iota(jnp.int32, LANES)
        m16 = rcols[0]
        for t in rcols[1:]:
            m16 = jnp.maximum(m16, t)
        for sh in (1, 2, 4, 8):
            m16 = jnp.maximum(m16, _lane_perm(m16, lane ^ sh))
        es = [jnp.exp(t - m16) for t in rcols]
        s16 = es[0]
        for t in es[1:]:
            s16 = s16 + t
        for sh in (1, 2, 4, 8):
            s16 = s16 + _lane_perm(s16, lane ^ sh)
        inv = 1.0 / s16
        for col in range(COLS):
            out_v[j, pl.ds(col * LANES, LANES)] = es[col] * inv

    def body(i, _):
        for b in range(NBUF):
            j = i * NBUF + b
            drain(j, b)
            nxt = j + NBUF

            @pl.when(nxt < SLOTS_PW)
            def _():
                issue(nxt, b)

            compute(j, b)
        return _

    lax.fori_loop(0, SLOTS_PW // NBUF, body, None)
    pltpu.sync_copy(out_v, out_hbm.at[pl.ds(wid * SLOTS_PW, SLOTS_PW)])


def _sc_encode(T, node_tbl, bn):
    mesh = plsc.VectorSubcoreMesh(core_axis_name="c", subcore_axis_name="s")
    return pl.kernel(
        _sc_body,
        out_type=jax.ShapeDtypeStruct((PAIRS * SPP, D), jnp.float32),
        mesh=mesh,
        compiler_params=pltpu.CompilerParams(use_tc_tiling_on_sc=True),
        scratch_types=[
            pltpu.VMEM((PPW, 1), jnp.int32),
            pltpu.VMEM((SLOTS_PW, D), jnp.int32),
            pltpu.VMEM((NBUF, ROWS, D), jnp.float32),
            pltpu.VMEM((SLOTS_PW, D), jnp.float32),
            pltpu.SemaphoreType.DMA,
        ] + [pltpu.SemaphoreType.DMA] * NBUF,
    )(T, node_tbl, bn.reshape(-1, 1))


def _post_body(enc_ref, U_ref, V_ref, W1_ref, b1_ref, W2_ref, b2_ref,
               out_ref):
    enc = enc_ref[...]
    e0 = enc[:, 0, :]
    nsum = jnp.sum(enc[:, 1:, :], axis=1)
    r = (lax.dot_general(e0, U_ref[...], (((1,), (1,)), ((), ())),
                         preferred_element_type=jnp.float32)
         + lax.dot_general(nsum, V_ref[...], (((1,), (1,)), ((), ())),
                           preferred_element_type=jnp.float32))
    ext = jax.nn.softmax(jax.nn.relu(r), axis=-1)
    e_i = ext[:B]
    e_j = ext[B:]
    third = jnp.concatenate([e_i * e_j, e_i + e_j], axis=-1)
    h = jax.nn.relu(
        lax.dot_general(third, W1_ref[...], (((1,), (1,)), ((), ())),
                        preferred_element_type=jnp.float32) + b1_ref[...])
    logits = lax.dot_general(h, W2_ref[...], (((1,), (1,)), ((), ())),
                             preferred_element_type=jnp.float32) + b2_ref[...]
    out_ref[...] = jax.nn.softmax(logits, axis=-1)


def _post(enc, U, V, W1, b1, W2, b2):
    return pl.pallas_call(
        _post_body,
        out_shape=jax.ShapeDtypeStruct((B, 2), jnp.float32),
    )(enc, U, V, W1, b1, W2, b2)


@jax.jit
def kernel(batch, internal_node_ids, internal_adj, ext_adj,
           Impact, W, M, U, V, W1, b1, W2, b2):
    K = Impact.shape[0]
    n = ext_adj.shape[0]
    batch = batch.astype(jnp.int32)
    ids32 = internal_node_ids.astype(jnp.int32)
    adj32 = internal_adj.reshape(n, P * A).astype(jnp.int32) + K
    ext32 = ext_adj.astype(jnp.int32)
    node_tbl = jnp.concatenate(
        [ids32, adj32, ext32,
         jnp.zeros((n, D - EXT_OFF - DEG), jnp.int32)], axis=1)
    bn = jnp.concatenate([batch[:, 0], batch[:, 1]], axis=0)

    T = _precompute_tables(Impact, jnp.stack([W, M]))
    enc = _sc_encode(T, node_tbl, bn)
    return _post(enc.reshape(PAIRS, SPP, D), U, V, W1,
                 b1.reshape(1, D), W2, b2.reshape(1, 2))


# speedup vs baseline: 1.2655x; 1.0907x over previous
import jax
import jax.numpy as jnp
from jax import lax
from jax.experimental import pallas as pl
from jax.experimental.pallas import tpu as pltpu
from jax.experimental.pallas import tpu_sc as plsc

D = 128
P = 8
A = 4
DEG = 16
B = 256

NC, NS = 2, 16
NW = NC * NS
PAIRS = 2 * B
PPW = PAIRS // NW
SPP = 1 + DEG
SLOTS_PW = PPW * SPP
ROWS = P * (1 + A)
EXT_OFF = ROWS
LANES = 16
COLS = D // LANES
NBUF = 2

KBLK = 5000


def _mm_body(imp_ref, wm_ref, out_ref):
    out_ref[...] = lax.dot_general(
        imp_ref[...], wm_ref[0],
        (((1,), (1,)), ((), ())),
        preferred_element_type=jnp.float32)


def _precompute_tables(Impact, WM):
    K = Impact.shape[0]
    nblk = K // KBLK
    return pl.pallas_call(
        _mm_body,
        grid=(2, nblk),
        in_specs=[
            pl.BlockSpec((KBLK, D), lambda i, j: (j, 0)),
            pl.BlockSpec((1, D, D), lambda i, j: (i, 0, 0)),
        ],
        out_specs=pl.BlockSpec((KBLK, D), lambda i, j: (i * nblk + j, 0)),
        out_shape=jax.ShapeDtypeStruct((2 * K, D), jnp.float32),
    )(Impact, WM)


def _lane_perm(x, idx):
    dnums = lax.GatherDimensionNumbers(
        offset_dims=(), collapsed_slice_dims=(0,), start_index_map=(0,))
    return lax.gather(x, idx[:, None], dnums, slice_sizes=(1,),
                      mode=lax.GatherScatterMode.PROMISE_IN_BOUNDS)


def _sc_body(T_hbm, node_hbm, bn2_hbm, out_hbm,
             selfs2_v, catidx_v, rows_v, out_v,
             sem_idx, *sems):
    c = lax.axis_index("c")
    s = lax.axis_index("s")
    wid = s * NC + c

    pltpu.sync_copy(bn2_hbm.at[pl.ds(wid * PPW, PPW)], selfs2_v)

    cps = [pltpu.async_copy(node_hbm.at[selfs2_v.at[g]],
                            catidx_v.at[pl.ds(g * SPP, 1)], sem_idx)
           for g in range(PPW)]
    for cp in cps:
        cp.wait()
    cps = [pltpu.async_copy(
               node_hbm.at[catidx_v.at[g * SPP, pl.ds(EXT_OFF, DEG)]],
               catidx_v.at[pl.ds(g * SPP + 1, DEG)], sem_idx)
           for g in range(PPW)]
    for cp in cps:
        cp.wait()

    def issue(j, b):
        pltpu.async_copy(T_hbm.at[catidx_v.at[j, pl.ds(0, ROWS)]],
                         rows_v.at[b], sems[b])

    def drain(j, b):
        pltpu.make_async_copy(T_hbm.at[catidx_v.at[j, pl.ds(0, ROWS)]],
                              rows_v.at[b], sems[b]).wait()

    for b in range(NBUF):
        issue(b, b)

    def compute(j, b):
        rcols = []
        for col in range(COLS):
            sl = pl.ds(col * LANES, LANES)
            r16 = None
            for p in range(P):
                acc = rows_v[b, p, sl]
                base = P + A * p
                for a in range(A):
                    acc = acc + rows_v[b, base + a, sl]
                sp = jnp.maximum(acc, 0.0)
                r16 = sp if r16 is None else r16 + sp
            rcols.append(r16)
        lane = lax.iota(jnp.int32, LANES)
        m16 = rcols[0]
        for t in rcols[1:]:
            m16 = jnp.maximum(m16, t)
        for sh in (1, 2, 4, 8):
            m16 = jnp.maximum(m16, _lane_perm(m16, lane ^ sh))
        es = [jnp.exp(t - m16) for t in rcols]
        s16 = es[0]
        for t in es[1:]:
            s16 = s16 + t
        for sh in (1, 2, 4, 8):
            s16 = s16 + _lane_perm(s16, lane ^ sh)
        inv = 1.0 / s16
        for col in range(COLS):
            out_v[j, pl.ds(col * LANES, LANES)] = es[col] * inv

    def body(i, _):
        for b in range(NBUF):
            j = i * NBUF + b
            drain(j, b)
            nxt = j + NBUF

            @pl.when(nxt < SLOTS_PW)
            def _():
                issue(nxt, b)

            compute(j, b)
        return _

    lax.fori_loop(0, SLOTS_PW // NBUF, body, None)
    pltpu.sync_copy(out_v, out_hbm.at[pl.ds(wid * SLOTS_PW, SLOTS_PW)])


def _sc_encode(T, node_tbl, bn):
    mesh = plsc.VectorSubcoreMesh(core_axis_name="c", subcore_axis_name="s")
    return pl.kernel(
        _sc_body,
        out_type=jax.ShapeDtypeStruct((PAIRS * SPP, D), jnp.float32),
        mesh=mesh,
        compiler_params=pltpu.CompilerParams(use_tc_tiling_on_sc=True),
        scratch_types=[
            pltpu.VMEM((PPW, 1), jnp.int32),
            pltpu.VMEM((SLOTS_PW, D), jnp.int32),
            pltpu.VMEM((NBUF, ROWS, D), jnp.float32),
            pltpu.VMEM((SLOTS_PW, D), jnp.float32),
            pltpu.SemaphoreType.DMA,
        ] + [pltpu.SemaphoreType.DMA] * NBUF,
    )(T, node_tbl, bn.reshape(-1, 1))


def _post_body(enc_ref, U_ref, V_ref, W1_ref, b1_ref, W2_ref, b2_ref,
               out_ref):
    enc = enc_ref[...]
    e0 = enc[:, 0, :]
    nsum = jnp.sum(enc[:, 1:, :], axis=1)
    r = (lax.dot_general(e0, U_ref[...], (((1,), (1,)), ((), ())),
                         preferred_element_type=jnp.float32)
         + lax.dot_general(nsum, V_ref[...], (((1,), (1,)), ((), ())),
                           preferred_element_type=jnp.float32))
    ext = jax.nn.softmax(jax.nn.relu(r), axis=-1)
    e_i = ext[:B]
    e_j = ext[B:]
    third = jnp.concatenate([e_i * e_j, e_i + e_j], axis=-1)
    h = jax.nn.relu(
        lax.dot_general(third, W1_ref[...], (((1,), (1,)), ((), ())),
                        preferred_element_type=jnp.float32) + b1_ref[...])
    logits = lax.dot_general(h, W2_ref[...], (((1,), (1,)), ((), ())),
                             preferred_element_type=jnp.float32) + b2_ref[...]
    out_ref[...] = jax.nn.softmax(logits, axis=-1)


def _post(enc, U, V, W1, b1, W2, b2):
    return pl.pallas_call(
        _post_body,
        out_shape=jax.ShapeDtypeStruct((B, 2), jnp.float32),
    )(enc, U, V, W1, b1, W2, b2)


@jax.jit
def kernel(batch, internal_node_ids, internal_adj, ext_adj,
           Impact, W, M, U, V, W1, b1, W2, b2):
    K = Impact.shape[0]
    n = ext_adj.shape[0]
    batch = batch.astype(jnp.int32)
    ids32 = internal_node_ids.astype(jnp.int32)
    adj32 = internal_adj.reshape(n, P * A).astype(jnp.int32) + K
    ext32 = ext_adj.astype(jnp.int32)
    node_tbl = jnp.concatenate(
        [ids32, adj32, ext32,
         jnp.zeros((n, D - EXT_OFF - DEG), jnp.int32)], axis=1)
    bn = jnp.concatenate([batch[:, 0], batch[:, 1]], axis=0)

    T = _precompute_tables(Impact, jnp.stack([W, M]))
    enc = _sc_encode(T, node_tbl, bn)
    return _post(enc.reshape(PAIRS, SPP, D), U, V, W1,
                 b1.reshape(1, D), W2, b2.reshape(1, 2))


# speedup vs baseline: 1.3107x; 1.0357x over previous
import jax
import jax.numpy as jnp
from jax import lax
from jax.experimental import pallas as pl
from jax.experimental.pallas import tpu as pltpu
from jax.experimental.pallas import tpu_sc as plsc

D = 128
P = 8
A = 4
DEG = 16
B = 256

NC, NS = 2, 16
NW = NC * NS
PAIRS = 2 * B
PPW = PAIRS // NW
SPP = 1 + DEG
SLOTS_PW = PPW * SPP
ROWS = P * (1 + A)
EXT_OFF = ROWS
LANES = 16
COLS = D // LANES
NBUF = 2

KBLK = 10000


def _mm_body(imp_ref, wm_ref, out_ref):
    out_ref[...] = lax.dot_general(
        imp_ref[...], wm_ref[0],
        (((1,), (1,)), ((), ())),
        preferred_element_type=jnp.float32)


def _precompute_tables(Impact, WM):
    K = Impact.shape[0]
    nblk = K // KBLK
    return pl.pallas_call(
        _mm_body,
        grid=(2, nblk),
        in_specs=[
            pl.BlockSpec((KBLK, D), lambda i, j: (j, 0)),
            pl.BlockSpec((1, D, D), lambda i, j: (i, 0, 0)),
        ],
        out_specs=pl.BlockSpec((KBLK, D), lambda i, j: (i * nblk + j, 0)),
        out_shape=jax.ShapeDtypeStruct((2 * K, D), jnp.float32),
    )(Impact, WM)


def _lane_perm(x, idx):
    dnums = lax.GatherDimensionNumbers(
        offset_dims=(), collapsed_slice_dims=(0,), start_index_map=(0,))
    return lax.gather(x, idx[:, None], dnums, slice_sizes=(1,),
                      mode=lax.GatherScatterMode.PROMISE_IN_BOUNDS)


def _sc_body(T_hbm, node_hbm, bn2_hbm, out_hbm,
             selfs2_v, catidx_v, rows_v, out_v,
             sem_idx, *sems):
    c = lax.axis_index("c")
    s = lax.axis_index("s")
    wid = s * NC + c

    pltpu.sync_copy(bn2_hbm.at[pl.ds(wid * PPW, PPW)], selfs2_v)

    cps = [pltpu.async_copy(node_hbm.at[selfs2_v.at[g]],
                            catidx_v.at[pl.ds(g * SPP, 1)], sem_idx)
           for g in range(PPW)]
    for cp in cps:
        cp.wait()
    cps = [pltpu.async_copy(
               node_hbm.at[catidx_v.at[g * SPP, pl.ds(EXT_OFF, DEG)]],
               catidx_v.at[pl.ds(g * SPP + 1, DEG)], sem_idx)
           for g in range(PPW)]
    for cp in cps:
        cp.wait()

    def issue(j, b):
        pltpu.async_copy(T_hbm.at[catidx_v.at[j, pl.ds(0, ROWS)]],
                         rows_v.at[b], sems[b])

    def drain(j, b):
        pltpu.make_async_copy(T_hbm.at[catidx_v.at[j, pl.ds(0, ROWS)]],
                              rows_v.at[b], sems[b]).wait()

    for b in range(NBUF):
        issue(b, b)

    def compute(j, b):
        rcols = []
        for col in range(COLS):
            sl = pl.ds(col * LANES, LANES)
            r16 = None
            for p in range(P):
                acc = rows_v[b, p, sl]
                base = P + A * p
                for a in range(A):
                    acc = acc + rows_v[b, base + a, sl]
                sp = jnp.maximum(acc, 0.0)
                r16 = sp if r16 is None else r16 + sp
            rcols.append(r16)
        lane = lax.iota(jnp.int32, LANES)
        m16 = rcols[0]
        for t in rcols[1:]:
            m16 = jnp.maximum(m16, t)
        for sh in (1, 2, 4, 8):
            m16 = jnp.maximum(m16, _lane_perm(m16, lane ^ sh))
        es = [jnp.exp(t - m16) for t in rcols]
        s16 = es[0]
        for t in es[1:]:
            s16 = s16 + t
        for sh in (1, 2, 4, 8):
            s16 = s16 + _lane_perm(s16, lane ^ sh)
        inv = 1.0 / s16
        for col in range(COLS):
            out_v[j, pl.ds(col * LANES, LANES)] = es[col] * inv

    def body(i, _):
        for b in range(NBUF):
            j = i * NBUF + b
            drain(j, b)
            nxt = j + NBUF

            @pl.when(nxt < SLOTS_PW)
            def _():
                issue(nxt, b)

            compute(j, b)
        return _

    lax.fori_loop(0, SLOTS_PW // NBUF, body, None)
    pltpu.sync_copy(out_v, out_hbm.at[pl.ds(wid * SLOTS_PW, SLOTS_PW)])


def _sc_encode(T, node_tbl, bn):
    mesh = plsc.VectorSubcoreMesh(core_axis_name="c", subcore_axis_name="s")
    return pl.kernel(
        _sc_body,
        out_type=jax.ShapeDtypeStruct((PAIRS * SPP, D), jnp.float32),
        mesh=mesh,
        compiler_params=pltpu.CompilerParams(use_tc_tiling_on_sc=True),
        scratch_types=[
            pltpu.VMEM((PPW, 1), jnp.int32),
            pltpu.VMEM((SLOTS_PW, D), jnp.int32),
            pltpu.VMEM((NBUF, ROWS, D), jnp.float32),
            pltpu.VMEM((SLOTS_PW, D), jnp.float32),
            pltpu.SemaphoreType.DMA,
        ] + [pltpu.SemaphoreType.DMA] * NBUF,
    )(T, node_tbl, bn.reshape(-1, 1))


def _post_body(enc_ref, U_ref, V_ref, W1_ref, b1_ref, W2_ref, b2_ref,
               out_ref):
    enc = enc_ref[...]
    e0 = enc[:, 0, :]
    nsum = jnp.sum(enc[:, 1:, :], axis=1)
    r = (lax.dot_general(e0, U_ref[...], (((1,), (1,)), ((), ())),
                         preferred_element_type=jnp.float32)
         + lax.dot_general(nsum, V_ref[...], (((1,), (1,)), ((), ())),
                           preferred_element_type=jnp.float32))
    ext = jax.nn.softmax(jax.nn.relu(r), axis=-1)
    e_i = ext[:B]
    e_j = ext[B:]
    third = jnp.concatenate([e_i * e_j, e_i + e_j], axis=-1)
    h = jax.nn.relu(
        lax.dot_general(third, W1_ref[...], (((1,), (1,)), ((), ())),
                        preferred_element_type=jnp.float32) + b1_ref[...])
    logits = lax.dot_general(h, W2_ref[...], (((1,), (1,)), ((), ())),
                             preferred_element_type=jnp.float32) + b2_ref[...]
    out_ref[...] = jax.nn.softmax(logits, axis=-1)


def _post(enc, U, V, W1, b1, W2, b2):
    return pl.pallas_call(
        _post_body,
        out_shape=jax.ShapeDtypeStruct((B, 2), jnp.float32),
    )(enc, U, V, W1, b1, W2, b2)


@jax.jit
def kernel(batch, internal_node_ids, internal_adj, ext_adj,
           Impact, W, M, U, V, W1, b1, W2, b2):
    K = Impact.shape[0]
    n = ext_adj.shape[0]
    batch = batch.astype(jnp.int32)
    ids32 = internal_node_ids.astype(jnp.int32)
    adj32 = internal_adj.reshape(n, P * A).astype(jnp.int32) + K
    ext32 = ext_adj.astype(jnp.int32)
    node_tbl = jnp.concatenate(
        [ids32, adj32, ext32,
         jnp.zeros((n, D - EXT_OFF - DEG), jnp.int32)], axis=1)
    bn = jnp.concatenate([batch[:, 0], batch[:, 1]], axis=0)

    T = _precompute_tables(Impact, jnp.stack([W, M]))
    enc = _sc_encode(T, node_tbl, bn)
    return _post(enc.reshape(PAIRS, SPP, D), U, V, W1,
                 b1.reshape(1, D), W2, b2.reshape(1, 2))
